# Initial kernel scaffold; baseline (speedup 1.0000x reference)
import jax
import jax.numpy as jnp
from jax.experimental import pallas as pl


def kernel(loc_p, loc_t, grid):
    raise NotImplementedError("write your pallas kernel here")


# speedup vs baseline: 1.3307x; 1.3307x over previous
import jax
import jax.numpy as jnp
from jax.experimental import pallas as pl
from jax.experimental.pallas import tpu as pltpu

_K = 10.0
_EPS = 1e-9


def _piou_body(locp_ref, loct_ref, grid_ref, out_ref):
    gx = grid_ref[:, 0:1]
    gy = grid_ref[:, 1:2]

    def weights(loc):
        cx = loc[0:1, :]
        cy = loc[1:2, :]
        w = loc[2:3, :]
        h = loc[3:4, :]
        th = loc[4:5, :]
        dx = gx - cx
        dy = gy - cy
        ct = jnp.cos(th)
        st = jnp.sin(th)
        dw = jnp.abs(dx * ct + dy * st)
        dh = jnp.abs(-dx * st + dy * ct)
        kw = jax.nn.sigmoid(-_K * (dw - 0.5 * w))
        kh = jax.nn.sigmoid(-_K * (dh - 0.5 * h))
        return kw * kh

    fp = weights(locp_ref[...])
    ft = weights(loct_ref[...])
    prod = fp * ft
    inter = jnp.sum(prod, axis=0)
    union = jnp.sum(fp, axis=0) + jnp.sum(ft, axis=0) - inter
    out_ref[...] = (inter / (union + _EPS))[None, None, :]


def kernel(loc_p, loc_t, grid):
    n = loc_p.shape[0]
    g = grid.shape[0]
    blk = 512
    n_pad = ((n + blk - 1) // blk) * blk
    pad = n_pad - n
    lp = jnp.pad(loc_p, ((0, pad), (0, 0))).T
    lt = jnp.pad(loc_t, ((0, pad), (0, 0))).T
    nblk = n_pad // blk

    out = pl.pallas_call(
        _piou_body,
        grid=(nblk,),
        in_specs=[
            pl.BlockSpec((5, blk), lambda i: (0, i)),
            pl.BlockSpec((5, blk), lambda i: (0, i)),
            pl.BlockSpec((g, 2), lambda i: (0, 0)),
        ],
        out_specs=pl.BlockSpec((1, 1, blk), lambda i: (i, 0, 0)),
        out_shape=jax.ShapeDtypeStruct((nblk, 1, blk), jnp.float32),
    )(lp, lt, grid)
    return out.reshape(-1)[:n]


# speedup vs baseline: 1.3983x; 1.0508x over previous
import jax
import jax.numpy as jnp
from jax.experimental import pallas as pl
from jax.experimental.pallas import tpu as pltpu

_K = 10.0
_EPS = 1e-9


def _piou_body(locp_ref, loct_ref, grid_ref, out_ref):
    gx = grid_ref[:, 0:1]
    gy = grid_ref[:, 1:2]

    def denom(loc):
        cx = loc[0:1, :]
        cy = loc[1:2, :]
        th = loc[4:5, :]
        kct = _K * jnp.cos(th)
        kst = _K * jnp.sin(th)
        kw2 = (0.5 * _K) * loc[2:3, :]
        kh2 = (0.5 * _K) * loc[3:4, :]
        dx = gx - cx
        dy = gy - cy
        a = jnp.abs(dx * kct + dy * kst) - kw2
        b = jnp.abs(dy * kct - dx * kst) - kh2
        ea = jnp.exp(jnp.minimum(a, 20.0))
        eb = jnp.exp(jnp.minimum(b, 20.0))
        return (1.0 + ea) * (1.0 + eb)

    dp = denom(locp_ref[...])
    dt = denom(loct_ref[...])
    r = 1.0 / (dp * dt)
    inter = jnp.sum(r, axis=0)
    union = jnp.sum((dp + dt - 1.0) * r, axis=0)
    out_ref[...] = (inter / (union + _EPS))[None, None, :]


def kernel(loc_p, loc_t, grid):
    n = loc_p.shape[0]
    g = grid.shape[0]
    blk = 512
    n_pad = ((n + blk - 1) // blk) * blk
    pad = n_pad - n
    lp = jnp.pad(loc_p, ((0, pad), (0, 0))).T
    lt = jnp.pad(loc_t, ((0, pad), (0, 0))).T
    nblk = n_pad // blk

    out = pl.pallas_call(
        _piou_body,
        grid=(nblk,),
        in_specs=[
            pl.BlockSpec((5, blk), lambda i: (0, i)),
            pl.BlockSpec((5, blk), lambda i: (0, i)),
            pl.BlockSpec((g, 2), lambda i: (0, 0)),
        ],
        out_specs=pl.BlockSpec((1, 1, blk), lambda i: (i, 0, 0)),
        out_shape=jax.ShapeDtypeStruct((nblk, 1, blk), jnp.float32),
    )(lp, lt, grid)
    return out.reshape(-1)[:n]


# speedup vs baseline: 1.5024x; 1.0745x over previous
import jax
import jax.numpy as jnp
from jax.experimental import pallas as pl
from jax.experimental.pallas import tpu as pltpu

_K = 10.0
_EPS = 1e-9


def _piou_body(locp_ref, loct_ref, grid_ref, out_ref):
    gx = grid_ref[:, 0:1]
    gy = grid_ref[:, 1:2]

    kl2 = _K * 1.4426950408889634

    def denom(loc):
        cx = loc[0:1, :]
        cy = loc[1:2, :]
        th = loc[4:5, :]
        kct = kl2 * jnp.cos(th)
        kst = kl2 * jnp.sin(th)
        kw2 = (0.5 * kl2) * loc[2:3, :]
        kh2 = (0.5 * kl2) * loc[3:4, :]
        dx = gx - cx
        dy = gy - cy
        a = jnp.abs(dx * kct + dy * kst) - kw2
        b = jnp.abs(dy * kct - dx * kst) - kh2
        ea = jnp.exp2(jnp.minimum(a, 29.0))
        eb = jnp.exp2(jnp.minimum(b, 29.0))
        return (1.0 + ea) * (1.0 + eb)

    dp = denom(locp_ref[...])
    dt = denom(loct_ref[...])
    r = 1.0 / (dp * dt)
    inter = jnp.sum(r, axis=0)
    union = jnp.sum((dp + dt - 1.0) * r, axis=0)
    out_ref[...] = (inter / (union + _EPS))[None, None, :]


def kernel(loc_p, loc_t, grid):
    n = loc_p.shape[0]
    g = grid.shape[0]
    blk = 512
    n_pad = ((n + blk - 1) // blk) * blk
    pad = n_pad - n
    lp = jnp.pad(loc_p, ((0, pad), (0, 0))).T
    lt = jnp.pad(loc_t, ((0, pad), (0, 0))).T
    nblk = n_pad // blk

    out = pl.pallas_call(
        _piou_body,
        grid=(nblk,),
        in_specs=[
            pl.BlockSpec((5, blk), lambda i: (0, i)),
            pl.BlockSpec((5, blk), lambda i: (0, i)),
            pl.BlockSpec((g, 2), lambda i: (0, 0)),
        ],
        out_specs=pl.BlockSpec((1, 1, blk), lambda i: (i, 0, 0)),
        out_shape=jax.ShapeDtypeStruct((nblk, 1, blk), jnp.float32),
    )(lp, lt, grid)
    return out.reshape(-1)[:n]
